# baseline (speedup 1.0000x reference)
import jax
import jax.numpy as jnp
from jax import lax
from jax.experimental import pallas as pl
from jax.experimental.pallas import tpu as pltpu
from jax.experimental.pallas import tpu_sc as plsc

N_ATOMS = 100000
NUM_TYPES = 64
L = 16
NUM_WORKERS = 32

CHUNK = 3136
LAST = N_ATOMS - (NUM_WORKERS - 1) * CHUNK


def _sc_body(energy_hbm, types_hbm, shifts_hbm, scales_hbm, out_hbm,
             shifts_v, scales_v, types_v, energy_v, out_v, sem_tab, sem, sem2):
    wid = lax.axis_index("s") * 2 + lax.axis_index("c")
    base = wid * CHUNK

    t1 = pltpu.async_copy(shifts_hbm, shifts_v, sem_tab)
    t2 = pltpu.async_copy(scales_hbm, scales_v, sem_tab)

    def work(size):
        h = size // 2
        c1 = pltpu.async_copy(types_hbm.at[pl.ds(base, h)],
                              types_v.at[pl.ds(0, h)], sem)
        c2 = pltpu.async_copy(energy_hbm.at[pl.ds(base, h)],
                              energy_v.at[pl.ds(0, h)], sem)
        c3 = pltpu.async_copy(types_hbm.at[pl.ds(base + h, h)],
                              types_v.at[pl.ds(h, h)], sem2)
        c4 = pltpu.async_copy(energy_hbm.at[pl.ds(base + h, h)],
                              energy_v.at[pl.ds(h, h)], sem2)
        t1.wait()
        t2.wait()
        c1.wait()
        c2.wait()

        @plsc.parallel_loop(0, h, L, unroll=8)
        def _(off):
            t = types_v[pl.ds(off, L)]
            sh = plsc.load_gather(shifts_v, [t])
            sc = plsc.load_gather(scales_v, [t])
            e = energy_v[pl.ds(off, L)]
            out_v[pl.ds(off, L)] = sh + sc * e

        o1 = pltpu.async_copy(out_v.at[pl.ds(0, h)],
                              out_hbm.at[pl.ds(base, h)], sem)
        c3.wait()
        c4.wait()

        @plsc.parallel_loop(0, h, L, unroll=8)
        def _(off):
            t = types_v[pl.ds(h + off, L)]
            sh = plsc.load_gather(shifts_v, [t])
            sc = plsc.load_gather(scales_v, [t])
            e = energy_v[pl.ds(h + off, L)]
            out_v[pl.ds(h + off, L)] = sh + sc * e

        o2 = pltpu.async_copy(out_v.at[pl.ds(h, h)],
                              out_hbm.at[pl.ds(base + h, h)], sem2)
        o1.wait()
        o2.wait()

    @pl.when(wid < NUM_WORKERS - 1)
    def _():
        work(CHUNK)

    @pl.when(wid == NUM_WORKERS - 1)
    def _():
        work(LAST)


@jax.jit
def _run(energy, types, shifts, scales):
    mesh = plsc.VectorSubcoreMesh(core_axis_name="c", subcore_axis_name="s")
    return pl.kernel(
        _sc_body,
        out_type=jax.ShapeDtypeStruct((N_ATOMS,), jnp.float32),
        mesh=mesh,
        compiler_params=pltpu.CompilerParams(needs_layout_passes=False),
        scratch_types=[
            pltpu.VMEM((NUM_TYPES,), jnp.float32),
            pltpu.VMEM((NUM_TYPES,), jnp.float32),
            pltpu.VMEM((CHUNK,), jnp.int32),
            pltpu.VMEM((CHUNK,), jnp.float32),
            pltpu.VMEM((CHUNK,), jnp.float32),
            pltpu.SemaphoreType.DMA,
            pltpu.SemaphoreType.DMA,
            pltpu.SemaphoreType.DMA,
        ],
    )(energy, types, shifts, scales)


def kernel(atomic_energy, atom_types, shifts, scales):
    energy = atomic_energy.astype(jnp.float32).reshape(-1)
    types = atom_types.astype(jnp.int32).reshape(-1)
    out = _run(energy, types, shifts, scales)
    return out.reshape(-1, 1)


# speedup vs baseline: 1.0124x; 1.0124x over previous
import jax
import jax.numpy as jnp
from jax import lax
from jax.experimental import pallas as pl
from jax.experimental.pallas import tpu as pltpu
from jax.experimental.pallas import tpu_sc as plsc

N_ATOMS = 100000
NUM_TYPES = 64
L = 16
NUM_WORKERS = 32

CHUNK = 3136
LAST = N_ATOMS - (NUM_WORKERS - 1) * CHUNK


def _sc_body(energy_hbm, types_hbm, shifts_hbm, scales_hbm, out_hbm,
             shifts_v, scales_v, types_v, energy_v, out_v, sem_tab, sem):
    wid = lax.axis_index("s") * 2 + lax.axis_index("c")
    base = wid * CHUNK

    t1 = pltpu.async_copy(shifts_hbm, shifts_v, sem_tab)
    t2 = pltpu.async_copy(scales_hbm, scales_v, sem_tab)

    def work(size):
        c1 = pltpu.async_copy(types_hbm.at[pl.ds(base, size)],
                              types_v.at[pl.ds(0, size)], sem)
        c2 = pltpu.async_copy(energy_hbm.at[pl.ds(base, size)],
                              energy_v.at[pl.ds(0, size)], sem)
        t1.wait()
        t2.wait()
        c1.wait()
        c2.wait()

        @plsc.parallel_loop(0, size, L, unroll=8)
        def _(off):
            t = types_v[pl.ds(off, L)]
            sh = plsc.load_gather(shifts_v, [t])
            sc = plsc.load_gather(scales_v, [t])
            e = energy_v[pl.ds(off, L)]
            out_v[pl.ds(off, L)] = sh + sc * e

        pltpu.sync_copy(out_v.at[pl.ds(0, size)],
                        out_hbm.at[pl.ds(base, size)])

    @pl.when(wid < NUM_WORKERS - 1)
    def _():
        work(CHUNK)

    @pl.when(wid == NUM_WORKERS - 1)
    def _():
        work(LAST)


@jax.jit
def _run(energy, types, shifts, scales):
    mesh = plsc.VectorSubcoreMesh(core_axis_name="c", subcore_axis_name="s")
    return pl.kernel(
        _sc_body,
        out_type=jax.ShapeDtypeStruct((N_ATOMS,), jnp.float32),
        mesh=mesh,
        compiler_params=pltpu.CompilerParams(
            needs_layout_passes=False,
            disable_bounds_checks=True,
            skip_device_barrier=True,
        ),
        scratch_types=[
            pltpu.VMEM((NUM_TYPES,), jnp.float32),
            pltpu.VMEM((NUM_TYPES,), jnp.float32),
            pltpu.VMEM((CHUNK,), jnp.int32),
            pltpu.VMEM((CHUNK,), jnp.float32),
            pltpu.VMEM((CHUNK,), jnp.float32),
            pltpu.SemaphoreType.DMA,
            pltpu.SemaphoreType.DMA,
        ],
    )(energy, types, shifts, scales)


def kernel(atomic_energy, atom_types, shifts, scales):
    energy = atomic_energy.astype(jnp.float32).reshape(-1)
    types = atom_types.astype(jnp.int32).reshape(-1)
    out = _run(energy, types, shifts, scales)
    return out.reshape(-1, 1)
